# Initial kernel scaffold; baseline (speedup 1.0000x reference)
import jax
import jax.numpy as jnp
from jax.experimental import pallas as pl


def kernel(q_shape_h, q_shape_w, relative_position_bias_table):
    raise NotImplementedError("write your pallas kernel here")


# speedup vs baseline: 23.8122x; 23.8122x over previous
import functools

import numpy as np
import jax
import jax.numpy as jnp
from jax import lax
from jax.experimental import pallas as pl
from jax.experimental.pallas import tpu as pltpu
from jax.experimental.pallas import tpu_sc as plsc

NUM_HEADS = 16
KV = 16
QH = 64
QW = 64
W_IN = 31
W_OUT = 79
TABLE_WORDS = NUM_HEADS * W_OUT * W_OUT

NC = 2
NS = 16
NW = NC * NS
ROWS_TOTAL = NUM_HEADS * QH * QW
ROWS_PER_W = ROWS_TOTAL // NW
BLK = 64
NBLK = ROWS_PER_W // BLK


def _cubic_kernel(x):
    out = ((1.5 * x - 2.5) * x) * x + 1.0
    out = np.where(x >= 1.0, ((-0.5 * x + 2.5) * x - 4.0) * x + 2.0, out)
    return np.where(x >= 2.0, 0.0, out)


def _weight_mat(in_size, out_size):
    inv_scale = in_size / out_size
    sample_f = (np.arange(out_size, dtype=np.float64) + 0.5) * inv_scale - 0.5
    x = np.abs(sample_f[None, :] - np.arange(in_size, dtype=np.float64)[:, None])
    w = _cubic_kernel(x)
    tot = np.sum(w, axis=0, keepdims=True)
    w = np.where(np.abs(tot) > 1000 * np.finfo(np.float32).eps,
                 w / np.where(tot != 0, tot, 1), 0)
    w = w * ((sample_f >= -0.5) & (sample_f <= in_size - 0.5))[None, :]
    return w.astype(np.float32)


_WFLIP = _weight_mat(W_IN, W_OUT)[:, ::-1]
_WT_CONST = np.ascontiguousarray(_WFLIP.T)
_WF_CONST = np.ascontiguousarray(_WFLIP)


def _resize_body(t_ref, wt_ref, wf_ref, out_ref):
    wt = wt_ref[...]
    wf = wf_ref[...]
    for h in range(NUM_HEADS):
        tmp = jnp.dot(wt, t_ref[h], preferred_element_type=jnp.float32)
        out_ref[h] = jnp.dot(tmp, wf, preferred_element_type=jnp.float32)


def _resize_tc(table_t):
    return pl.pallas_call(
        _resize_body,
        out_shape=jax.ShapeDtypeStruct((NUM_HEADS, W_OUT, W_OUT), jnp.float32),
    )(table_t, jnp.asarray(_WT_CONST), jnp.asarray(_WF_CONST))


def _sc_gather_body(f_hbm, out_hbm, f_v, buf, sem):
    wid = lax.axis_index("s") * NC + lax.axis_index("c")
    row_base = wid * ROWS_PER_W
    h = row_base // (QH * QW)
    q_base = row_base % (QH * QW)
    c0 = h * (W_OUT * W_OUT) + (QH - 1) * W_OUT + (QW - 1)

    pltpu.sync_copy(f_hbm, f_v)

    def block_body(b, _):
        def row_body(i, _):
            q = q_base + b * BLK + i
            qi = q // QW
            qj = q % QW
            base = c0 - 79 * qi - qj
            for ki in range(KV):
                buf[i, pl.ds(16 * ki, 16)] = f_v[pl.ds(base + 79 * ki, 16)]
            return 0
        lax.fori_loop(0, BLK, row_body, 0)
        pltpu.sync_copy(buf, out_hbm.at[pl.ds(row_base + b * BLK, BLK)])
        return 0

    lax.fori_loop(0, NBLK, block_body, 0)


@functools.lru_cache(maxsize=None)
def _make_sc_gather():
    return pl.kernel(
        _sc_gather_body,
        out_type=jax.ShapeDtypeStruct((ROWS_TOTAL, KV * KV), jnp.float32),
        mesh=plsc.VectorSubcoreMesh(core_axis_name="c", subcore_axis_name="s",
                                    num_cores=NC, num_subcores=NS),
        scratch_types=[
            pltpu.VMEM((TABLE_WORDS,), jnp.float32),
            pltpu.VMEM((BLK, KV * KV), jnp.float32),
            pltpu.SemaphoreType.DMA,
        ],
    )


def kernel(q_shape_h, q_shape_w, relative_position_bias_table):
    t = relative_position_bias_table.reshape(W_IN, W_IN, NUM_HEADS)
    t = jnp.transpose(t, (2, 0, 1))
    f = _resize_tc(t).reshape(TABLE_WORDS)
    out = _make_sc_gather()(f)
    return out.reshape(NUM_HEADS, QH * QW, KV * KV)


# speedup vs baseline: 27.1591x; 1.1406x over previous
import functools

import numpy as np
import jax
import jax.numpy as jnp
from jax import lax
from jax.experimental import pallas as pl
from jax.experimental.pallas import tpu as pltpu
from jax.experimental.pallas import tpu_sc as plsc

NUM_HEADS = 16
KV = 16
QH = 64
QW = 64
W_IN = 31
W_OUT = 79
TABLE_WORDS = NUM_HEADS * W_OUT * W_OUT

NC = 2
NS = 16
NW = NC * NS
ROWS_TOTAL = NUM_HEADS * QH * QW
ROWS_PER_W = ROWS_TOTAL // NW
BLK = 32
NBLK = ROWS_PER_W // BLK


def _cubic_kernel(x):
    out = ((1.5 * x - 2.5) * x) * x + 1.0
    out = np.where(x >= 1.0, ((-0.5 * x + 2.5) * x - 4.0) * x + 2.0, out)
    return np.where(x >= 2.0, 0.0, out)


def _weight_mat(in_size, out_size):
    inv_scale = in_size / out_size
    sample_f = (np.arange(out_size, dtype=np.float64) + 0.5) * inv_scale - 0.5
    x = np.abs(sample_f[None, :] - np.arange(in_size, dtype=np.float64)[:, None])
    w = _cubic_kernel(x)
    tot = np.sum(w, axis=0, keepdims=True)
    w = np.where(np.abs(tot) > 1000 * np.finfo(np.float32).eps,
                 w / np.where(tot != 0, tot, 1), 0)
    w = w * ((sample_f >= -0.5) & (sample_f <= in_size - 0.5))[None, :]
    return w.astype(np.float32)


_WFLIP = _weight_mat(W_IN, W_OUT)[:, ::-1]
_WT_CONST = np.ascontiguousarray(_WFLIP.T)
_WF_CONST = np.ascontiguousarray(_WFLIP)


def _resize_body(t_ref, wt_ref, wf_ref, out_ref):
    wt = wt_ref[...]
    wf = wf_ref[...]
    for h in range(NUM_HEADS):
        tmp = jnp.dot(wt, t_ref[h], preferred_element_type=jnp.float32)
        out_ref[h] = jnp.dot(tmp, wf, preferred_element_type=jnp.float32)


def _resize_tc(table_t):
    return pl.pallas_call(
        _resize_body,
        out_shape=jax.ShapeDtypeStruct((NUM_HEADS, W_OUT, W_OUT), jnp.float32),
    )(table_t, jnp.asarray(_WT_CONST), jnp.asarray(_WF_CONST))


def _sc_gather_body(f_hbm, out_hbm, f_v, buf0, buf1, sem0, sem1):
    wid = lax.axis_index("s") * NC + lax.axis_index("c")
    row_base = wid * ROWS_PER_W
    h = row_base // (QH * QW)
    qi_base = (row_base % (QH * QW)) // QW
    c00 = h * (W_OUT * W_OUT) + (QH - 1) * W_OUT + (QW - 1)

    pltpu.sync_copy(f_hbm, f_v)

    def fill(buf, b2, qj0):
        base_row = c00 - 79 * (qi_base + b2) - qj0

        def row(i, _):
            base = base_row - i
            for ki in range(KV):
                buf[i, pl.ds(16 * ki, 16)] = f_v[pl.ds(base + 79 * ki, 16)]
            return 0

        lax.fori_loop(0, BLK, row, 0, unroll=2)

    def dst(b):
        return out_hbm.at[pl.ds(row_base + b * BLK, BLK)]

    def b2_body(b2, _):
        @pl.when(b2 > 0)
        def _():
            pltpu.make_async_copy(buf0, dst(0), sem0).wait()
        fill(buf0, b2, 0)
        pltpu.async_copy(buf0, dst(2 * b2), sem0)

        @pl.when(b2 > 0)
        def _():
            pltpu.make_async_copy(buf1, dst(0), sem1).wait()
        fill(buf1, b2, BLK)
        pltpu.async_copy(buf1, dst(2 * b2 + 1), sem1)
        return 0

    lax.fori_loop(0, NBLK // 2, b2_body, 0)
    pltpu.make_async_copy(buf0, dst(0), sem0).wait()
    pltpu.make_async_copy(buf1, dst(0), sem1).wait()


@functools.lru_cache(maxsize=None)
def _make_sc_gather():
    return pl.kernel(
        _sc_gather_body,
        out_type=jax.ShapeDtypeStruct((ROWS_TOTAL, KV * KV), jnp.float32),
        mesh=plsc.VectorSubcoreMesh(core_axis_name="c", subcore_axis_name="s",
                                    num_cores=NC, num_subcores=NS),
        scratch_types=[
            pltpu.VMEM((TABLE_WORDS,), jnp.float32),
            pltpu.VMEM((BLK, KV * KV), jnp.float32),
            pltpu.VMEM((BLK, KV * KV), jnp.float32),
            pltpu.SemaphoreType.DMA,
            pltpu.SemaphoreType.DMA,
        ],
    )


def kernel(q_shape_h, q_shape_w, relative_position_bias_table):
    t = relative_position_bias_table.reshape(W_IN, W_IN, NUM_HEADS)
    t = jnp.transpose(t, (2, 0, 1))
    f = _resize_tc(t).reshape(TABLE_WORDS)
    out = _make_sc_gather()(f)
    return out.reshape(NUM_HEADS, QH * QW, KV * KV)


# speedup vs baseline: 32.8000x; 1.2077x over previous
import functools

import numpy as np
import jax
import jax.numpy as jnp
from jax import lax
from jax.experimental import pallas as pl
from jax.experimental.pallas import tpu as pltpu
from jax.experimental.pallas import tpu_sc as plsc

NUM_HEADS = 16
KV = 16
QH = 64
QW = 64
W_IN = 31
W_OUT = 79

NC = 2
NS = 16
NW = NC * NS
ROWS_TOTAL = NUM_HEADS * QH * QW
ROWS_PER_W = ROWS_TOTAL // NW
BLK = 128
NBLK = ROWS_PER_W // BLK


def _cubic_kernel(x):
    out = ((1.5 * x - 2.5) * x) * x + 1.0
    out = np.where(x >= 1.0, ((-0.5 * x + 2.5) * x - 4.0) * x + 2.0, out)
    return np.where(x >= 2.0, 0.0, out)


def _weight_mat(in_size, out_size):
    inv_scale = in_size / out_size
    sample_f = (np.arange(out_size, dtype=np.float64) + 0.5) * inv_scale - 0.5
    x = np.abs(sample_f[None, :] - np.arange(in_size, dtype=np.float64)[:, None])
    w = _cubic_kernel(x)
    tot = np.sum(w, axis=0, keepdims=True)
    w = np.where(np.abs(tot) > 1000 * np.finfo(np.float32).eps,
                 w / np.where(tot != 0, tot, 1), 0)
    w = w * ((sample_f >= -0.5) & (sample_f <= in_size - 0.5))[None, :]
    return w.astype(np.float32)


_WFLIP = _weight_mat(W_IN, W_OUT)[:, ::-1]
_WT_CONST = np.ascontiguousarray(_WFLIP.T)
_WF_CONST = np.ascontiguousarray(_WFLIP)


W_PAD = 80


def _resize_body(t_ref, wt_ref, wf_ref, out_ref):
    wt = wt_ref[...]
    wf = wf_ref[...]
    for h in range(NUM_HEADS):
        tmp = jnp.dot(wt, t_ref[h], preferred_element_type=jnp.float32)
        out_ref[h, 0:W_OUT, :] = jnp.dot(tmp, wf,
                                         preferred_element_type=jnp.float32)
        out_ref[h, W_OUT:W_PAD, :] = jnp.zeros((W_PAD - W_OUT, W_OUT),
                                               jnp.float32)


def _resize_tc(table_t):
    return pl.pallas_call(
        _resize_body,
        out_shape=jax.ShapeDtypeStruct((NUM_HEADS, W_PAD, W_OUT), jnp.float32),
    )(table_t, jnp.asarray(_WT_CONST), jnp.asarray(_WF_CONST))


N_QI = ROWS_PER_W // QW
QI_PER_BLK = BLK // QW
BAND = N_QI + KV


def _sc_gather_body(f_hbm, out_hbm, band, buf0, buf1, sem0, sem1):
    wid = lax.axis_index("s") * NC + lax.axis_index("c")
    row_base = wid * ROWS_PER_W
    h = row_base // (QH * QW)
    qi_base = (row_base % (QH * QW)) // QW
    band_start = pl.multiple_of(h * W_PAD + (N_QI - qi_base), 8)
    pltpu.sync_copy(f_hbm.at[pl.ds(band_start, BAND)], band)

    def fill(buf, b):
        r_base0 = (N_QI - 1) - b * QI_PER_BLK

        def row(i, _):
            cj = (QW - 1) - i
            vals = [band[r_base0 + ki, pl.ds(cj, 16)] for ki in range(KV)]
            for ki in range(KV):
                buf[pl.ds(i * (KV * KV) + 16 * ki, 16)] = vals[ki]
            for a in range(1, QI_PER_BLK):
                vals = [band[r_base0 - a, pl.ds(cj, 16)]] + vals[:KV - 1]
                off = (a * QW + i) * (KV * KV)
                for ki in range(KV):
                    buf[pl.ds(off + 16 * ki, 16)] = vals[ki]
            return 0

        lax.fori_loop(0, QW, row, 0, unroll=4)

    NSPLIT = 4
    PIECE = BLK * KV * KV // NSPLIT

    def fire(buf, b, sem):
        for p in range(NSPLIT):
            pltpu.async_copy(
                buf.at[pl.ds(p * PIECE, PIECE)],
                out_hbm.at[pl.ds((row_base + b * BLK) * (KV * KV) + p * PIECE,
                                 PIECE)],
                sem)

    def drain(buf, sem):
        for p in range(NSPLIT):
            pltpu.make_async_copy(
                buf.at[pl.ds(0, PIECE)],
                out_hbm.at[pl.ds(row_base * (KV * KV), PIECE)],
                sem).wait()

    def b2_body(b2, _):
        @pl.when(b2 > 0)
        def _():
            drain(buf0, sem0)
        fill(buf0, 2 * b2)
        fire(buf0, 2 * b2, sem0)

        @pl.when(b2 > 0)
        def _():
            drain(buf1, sem1)
        fill(buf1, 2 * b2 + 1)
        fire(buf1, 2 * b2 + 1, sem1)
        return 0

    lax.fori_loop(0, NBLK // 2, b2_body, 0)
    drain(buf0, sem0)
    drain(buf1, sem1)


@functools.lru_cache(maxsize=None)
def _make_sc_gather():
    return pl.kernel(
        _sc_gather_body,
        out_type=jax.ShapeDtypeStruct((ROWS_TOTAL * KV * KV,), jnp.float32),
        mesh=plsc.VectorSubcoreMesh(core_axis_name="c", subcore_axis_name="s",
                                    num_cores=NC, num_subcores=NS),
        scratch_types=[
            pltpu.VMEM((BAND, W_OUT), jnp.float32),
            pltpu.VMEM((BLK * KV * KV,), jnp.float32),
            pltpu.VMEM((BLK * KV * KV,), jnp.float32),
            pltpu.SemaphoreType.DMA,
            pltpu.SemaphoreType.DMA,
        ],
    )


def kernel(q_shape_h, q_shape_w, relative_position_bias_table):
    t = relative_position_bias_table.reshape(W_IN, W_IN, NUM_HEADS)
    t = jnp.transpose(t, (2, 0, 1))
    f = _resize_tc(t).reshape(NUM_HEADS * W_PAD, W_OUT)
    out = _make_sc_gather()(f)
    return out.reshape(NUM_HEADS, QH * QW, KV * KV)
